# baseline (speedup 1.0000x reference)
import functools

import jax
import jax.numpy as jnp
from jax import lax
from jax.experimental import pallas as pl
from jax.experimental.pallas import tpu as pltpu
from jax.experimental.pallas import tpu_sc as plsc

IN_F = 32768
KSEL = 8192
BATCH = 128

L = 16
NC = 2
NS = 16
NBINS = 256
SHIFTS = (0, 8, 16, 24)
CHUNK = IN_F // NS
ROWS_PER_TILE = BATCH // (NC * NS)


def _key_body(s_ref, k_ref):
    s = s_ref[...]
    m = jnp.max(s)
    u = jnp.exp(s - m)
    p = u / jnp.sum(u)
    kb = lax.bitcast_convert_type(p, jnp.int32)
    k_ref[...] = jnp.bitwise_not(kb)


def _make_keys(score, interpret=False):
    s2 = score.reshape(256, 128)
    keys = pl.pallas_call(
        _key_body,
        out_shape=jax.ShapeDtypeStruct((256, 128), jnp.int32),
        interpret=interpret,
    )(s2)
    return keys.reshape(IN_F)


def _sc_body(keys_hbm, x_hbm, out_hbm,
             ck, cp, pos, digs, rcs, lst, lh, offs, ht, rowA, rowB, idxb,
             oA, oB, spKA, spKB, spPA, spPB, sph, sem, semr, semw):
    cid = lax.axis_index("c")
    sid = lax.axis_index("s")
    iota = lax.iota(jnp.int32, L)
    t0 = sid * CHUNK
    wid = cid * NS + sid
    rows = (rowA, rowB)
    obufs = (oA, oB)

    pre0 = pltpu.async_copy(x_hbm.at[wid * ROWS_PER_TILE], rowA, semr)
    pre1 = pltpu.async_copy(x_hbm.at[wid * ROWS_PER_TILE + 1], rowB, semr)

    for pno, shift in enumerate(SHIFTS):
        srcK = (None, spKB, spKA, spKB)[pno]
        srcP = (None, spPB, spPA, spPB)[pno]
        dstK = (spKB, spKA, spKB, spKA)[pno]
        dstP = (spPB, spPA, spPB, spPA)[pno]

        if pno == 0:
            pltpu.sync_copy(keys_hbm.at[pl.ds(t0, CHUNK)], ck)
        else:
            pltpu.sync_copy(srcK.at[pl.ds(t0, CHUNK)], ck)
            pltpu.sync_copy(srcP.at[pl.ds(t0, CHUNK)], cp)

        zv = jnp.zeros((L,), jnp.int32)
        for j in range(NBINS // L):
            lh[pl.ds(j * L, L)] = zv

        def _aux(j, _):
            for u in range(4):
                o = j * 4 * L + u * L
                k = ck[pl.ds(o, L)]
                dig = lax.shift_right_logical(k, shift) & (NBINS - 1)
                rc, last = plsc.scan_count(dig)
                digs[pl.ds(o, L)] = dig
                rcs[pl.ds(o, L)] = rc
                lst[pl.ds(o, L)] = jnp.where(last, 1, 0)
                plsc.addupdate_scatter(lh, [dig], rc, mask=last)
            return 0
        lax.fori_loop(0, CHUNK // (4 * L), _aux, 0)

        pltpu.sync_copy(lh, sph.at[pl.ds(sid * NBINS, NBINS)])
        plsc.subcore_barrier()
        pltpu.sync_copy(sph, ht)

        def _offsets(g, carry):
            total = jnp.zeros((L,), jnp.int32)
            before = jnp.zeros((L,), jnp.int32)
            for tp in range(NS):
                v = ht[pl.ds(tp * NBINS + g * L, L)]
                total = total + v
                before = before + v * jnp.where(tp < sid, 1, 0)
            ex = plsc.cumsum(total) - total + carry
            offs[pl.ds(g * L, L)] = ex + before
            return carry + jnp.sum(total)
        lax.fori_loop(0, NBINS // L, _offsets, jnp.int32(0))

        def _pos(j, _):
            for u in range(4):
                o = j * 4 * L + u * L
                dig = digs[pl.ds(o, L)]
                rc = rcs[pl.ds(o, L)]
                last = lst[pl.ds(o, L)] > 0
                base = plsc.load_gather(offs, [dig])
                pos[pl.ds(o, L)] = base + rc - 1
                plsc.store_scatter(offs, [dig], base + rc, mask=last)
                if pno == 0:
                    cp[pl.ds(o, L)] = t0 + o + iota
            return 0
        lax.fori_loop(0, CHUNK // (4 * L), _pos, 0)

        c2 = pltpu.async_copy(cp, dstP.at[pos], sem)
        if pno < len(SHIFTS) - 1:
            c1 = pltpu.async_copy(ck, dstK.at[pos], sem)
            c1.wait()
        c2.wait()
        plsc.subcore_barrier()

    pltpu.sync_copy(spPA.at[pl.ds(0, KSEL)], idxb)
    cps = [pre0, pre1, None, None]
    outs = [None] * ROWS_PER_TILE
    for rr in range(ROWS_PER_TILE):
        b = wid * ROWS_PER_TILE + rr
        cps[rr].wait()
        if rr >= 2:
            outs[rr - 2].wait()
        rowbuf = rows[rr % 2]
        obuf = obufs[rr % 2]

        def _gather(j, _):
            for u in range(4):
                o = j * 4 * L + u * L
                idxv = idxb[pl.ds(o, L)]
                obuf[pl.ds(o, L)] = plsc.load_gather(rowbuf, [idxv])
            return 0
        lax.fori_loop(0, KSEL // (4 * L), _gather, 0)
        outs[rr] = pltpu.async_copy(obuf, out_hbm.at[b], semw)
        if rr + 2 < ROWS_PER_TILE:
            cps[rr + 2] = pltpu.async_copy(x_hbm.at[b + 2], rowbuf, semr)
    for rr in range(max(0, ROWS_PER_TILE - 2), ROWS_PER_TILE):
        outs[rr].wait()


def _make_sc_call(interpret=False):
    mesh = plsc.VectorSubcoreMesh(core_axis_name="c", subcore_axis_name="s",
                                  num_cores=NC, num_subcores=NS)
    return pl.kernel(
        _sc_body,
        out_type=jax.ShapeDtypeStruct((BATCH, KSEL), jnp.float32),
        mesh=mesh,
        scratch_types=[
            pltpu.VMEM((CHUNK,), jnp.int32),
            pltpu.VMEM((CHUNK,), jnp.int32),
            pltpu.VMEM((CHUNK,), jnp.int32),
            pltpu.VMEM((CHUNK,), jnp.int32),
            pltpu.VMEM((CHUNK,), jnp.int32),
            pltpu.VMEM((CHUNK,), jnp.int32),
            pltpu.VMEM((NBINS,), jnp.int32),
            pltpu.VMEM((NBINS,), jnp.int32),
            pltpu.VMEM((NS * NBINS,), jnp.int32),
            pltpu.VMEM((IN_F,), jnp.float32),
            pltpu.VMEM((IN_F,), jnp.float32),
            pltpu.VMEM((KSEL,), jnp.int32),
            pltpu.VMEM((KSEL,), jnp.float32),
            pltpu.VMEM((KSEL,), jnp.float32),
            pltpu.VMEM_SHARED((IN_F,), jnp.int32),
            pltpu.VMEM_SHARED((IN_F,), jnp.int32),
            pltpu.VMEM_SHARED((IN_F,), jnp.int32),
            pltpu.VMEM_SHARED((IN_F,), jnp.int32),
            pltpu.VMEM_SHARED((NS * NBINS,), jnp.int32),
            pltpu.SemaphoreType.DMA,
            pltpu.SemaphoreType.DMA,
            pltpu.SemaphoreType.DMA,
        ],
        compiler_params=pltpu.CompilerParams(needs_layout_passes=False),
        interpret=interpret,
    )


def kernel(x, score):
    keys = _make_keys(score)
    return _make_sc_call()(keys, x)


# speedup vs baseline: 1.0502x; 1.0502x over previous
import functools

import jax
import jax.numpy as jnp
from jax import lax
from jax.experimental import pallas as pl
from jax.experimental.pallas import tpu as pltpu
from jax.experimental.pallas import tpu_sc as plsc

IN_F = 32768
KSEL = 8192
BATCH = 128

L = 16
NC = 2
NS = 16
NBINS = 256
SHIFTS = (0, 8, 16, 24)
CHUNK = IN_F // NS
ROWS_PER_TILE = BATCH // (NC * NS)


def _key_body(s_ref, k_ref):
    s = s_ref[...]
    m = jnp.max(s)
    u = jnp.exp(s - m)
    p = u / jnp.sum(u)
    kb = lax.bitcast_convert_type(p, jnp.int32)
    k_ref[...] = jnp.bitwise_not(kb)


def _make_keys(score, interpret=False):
    s2 = score.reshape(256, 128)
    keys = pl.pallas_call(
        _key_body,
        out_shape=jax.ShapeDtypeStruct((256, 128), jnp.int32),
        interpret=interpret,
    )(s2)
    return keys.reshape(IN_F)


def _sc_body(keys_hbm, x_hbm, out_hbm,
             ck, cp, pos, digs, rcs, lst, lh, offs, ht, rowA, rowB, idxb,
             oA, oB, spKA, spKB, spPA, spPB, sph, sem, semr, semw):
    cid = lax.axis_index("c")
    sid = lax.axis_index("s")
    iota = lax.iota(jnp.int32, L)
    t0 = sid * CHUNK
    wid = cid * NS + sid
    rows = (rowA, rowB)
    obufs = (oA, oB)

    pre0 = pltpu.async_copy(x_hbm.at[wid * ROWS_PER_TILE], rowA, semr)
    pre1 = pltpu.async_copy(x_hbm.at[wid * ROWS_PER_TILE + 1], rowB, semr)

    for pno, shift in enumerate(SHIFTS):
        srcK = (None, spKB, spKA, spKB)[pno]
        srcP = (None, spPB, spPA, spPB)[pno]
        dstK = (spKB, spKA, spKB, spKA)[pno]
        dstP = (spPB, spPA, spPB, spPA)[pno]

        if pno == 0:
            pltpu.sync_copy(keys_hbm.at[pl.ds(t0, CHUNK)], ck)
        else:
            pltpu.sync_copy(srcK.at[pl.ds(t0, CHUNK)], ck)
            pltpu.sync_copy(srcP.at[pl.ds(t0, CHUNK)], cp)

        def _aux(j, _):
            for u in range(4):
                o = j * 4 * L + u * L
                k = ck[pl.ds(o, L)]
                dig = lax.shift_right_logical(k, shift) & (NBINS - 1)
                rc, last = plsc.scan_count(dig)
                digs[pl.ds(o, L)] = dig
                rcs[pl.ds(o, L)] = rc
                lst[pl.ds(o, L)] = jnp.where(last, 1, 0)
            return 0
        lax.fori_loop(0, CHUNK // (4 * L), _aux, 0)

        def _zero(j, _):
            lh[pl.ds(j * L, L)] = jnp.zeros((L,), jnp.int32)
            return 0
        lax.fori_loop(0, NBINS // L, _zero, 0)

        def _hist(j, _):
            for u in range(4):
                o = j * 4 * L + u * L
                dig = digs[pl.ds(o, L)]
                rc = rcs[pl.ds(o, L)]
                last = lst[pl.ds(o, L)] > 0
                plsc.addupdate_scatter(lh, [dig], rc, mask=last)
            return 0
        lax.fori_loop(0, CHUNK // (4 * L), _hist, 0)

        pltpu.sync_copy(lh, sph.at[pl.ds(sid * NBINS, NBINS)])
        plsc.subcore_barrier()
        pltpu.sync_copy(sph, ht)

        def _offsets(g, carry):
            total = jnp.zeros((L,), jnp.int32)
            before = jnp.zeros((L,), jnp.int32)
            for tp in range(NS):
                v = ht[pl.ds(tp * NBINS + g * L, L)]
                total = total + v
                before = before + v * jnp.where(tp < sid, 1, 0)
            ex = plsc.cumsum(total) - total + carry
            offs[pl.ds(g * L, L)] = ex + before
            return carry + jnp.sum(total)
        lax.fori_loop(0, NBINS // L, _offsets, jnp.int32(0))

        def _pos(j, _):
            for u in range(4):
                o = j * 4 * L + u * L
                dig = digs[pl.ds(o, L)]
                rc = rcs[pl.ds(o, L)]
                last = lst[pl.ds(o, L)] > 0
                base = plsc.load_gather(offs, [dig])
                pos[pl.ds(o, L)] = base + rc - 1
                plsc.store_scatter(offs, [dig], base + rc, mask=last)
                if pno == 0:
                    cp[pl.ds(o, L)] = t0 + o + iota
            return 0
        lax.fori_loop(0, CHUNK // (4 * L), _pos, 0)

        c2 = pltpu.async_copy(cp, dstP.at[pos], sem)
        if pno < len(SHIFTS) - 1:
            c1 = pltpu.async_copy(ck, dstK.at[pos], sem)
            c1.wait()
        c2.wait()
        plsc.subcore_barrier()

    pltpu.sync_copy(spPA.at[pl.ds(0, KSEL)], idxb)
    cps = [pre0, pre1, None, None]
    outs = [None] * ROWS_PER_TILE
    for rr in range(ROWS_PER_TILE):
        b = wid * ROWS_PER_TILE + rr
        cps[rr].wait()
        if rr >= 2:
            outs[rr - 2].wait()
        rowbuf = rows[rr % 2]
        obuf = obufs[rr % 2]

        def _gather(j, _):
            for u in range(4):
                o = j * 4 * L + u * L
                idxv = idxb[pl.ds(o, L)]
                obuf[pl.ds(o, L)] = plsc.load_gather(rowbuf, [idxv])
            return 0
        lax.fori_loop(0, KSEL // (4 * L), _gather, 0)
        outs[rr] = pltpu.async_copy(obuf, out_hbm.at[b], semw)
        if rr + 2 < ROWS_PER_TILE:
            cps[rr + 2] = pltpu.async_copy(x_hbm.at[b + 2], rowbuf, semr)
    for rr in range(max(0, ROWS_PER_TILE - 2), ROWS_PER_TILE):
        outs[rr].wait()


def _make_sc_call(interpret=False):
    mesh = plsc.VectorSubcoreMesh(core_axis_name="c", subcore_axis_name="s",
                                  num_cores=NC, num_subcores=NS)
    return pl.kernel(
        _sc_body,
        out_type=jax.ShapeDtypeStruct((BATCH, KSEL), jnp.float32),
        mesh=mesh,
        scratch_types=[
            pltpu.VMEM((CHUNK,), jnp.int32),
            pltpu.VMEM((CHUNK,), jnp.int32),
            pltpu.VMEM((CHUNK,), jnp.int32),
            pltpu.VMEM((CHUNK,), jnp.int32),
            pltpu.VMEM((CHUNK,), jnp.int32),
            pltpu.VMEM((CHUNK,), jnp.int32),
            pltpu.VMEM((NBINS,), jnp.int32),
            pltpu.VMEM((NBINS,), jnp.int32),
            pltpu.VMEM((NS * NBINS,), jnp.int32),
            pltpu.VMEM((IN_F,), jnp.float32),
            pltpu.VMEM((IN_F,), jnp.float32),
            pltpu.VMEM((KSEL,), jnp.int32),
            pltpu.VMEM((KSEL,), jnp.float32),
            pltpu.VMEM((KSEL,), jnp.float32),
            pltpu.VMEM_SHARED((IN_F,), jnp.int32),
            pltpu.VMEM_SHARED((IN_F,), jnp.int32),
            pltpu.VMEM_SHARED((IN_F,), jnp.int32),
            pltpu.VMEM_SHARED((IN_F,), jnp.int32),
            pltpu.VMEM_SHARED((NS * NBINS,), jnp.int32),
            pltpu.SemaphoreType.DMA,
            pltpu.SemaphoreType.DMA,
            pltpu.SemaphoreType.DMA,
        ],
        compiler_params=pltpu.CompilerParams(needs_layout_passes=False),
        interpret=interpret,
    )


def kernel(x, score):
    keys = _make_keys(score)
    return _make_sc_call()(keys, x)
